# baseline (speedup 1.0000x reference)
import functools

import jax
import jax.numpy as jnp
from jax import lax
from jax.experimental import pallas as pl
from jax.experimental.pallas import tpu as pltpu
from jax.experimental.pallas import tpu_sc as plsc

N = 10000
B = 8
E = 160000
BE = 1024
E_PAD = 163840
BN = 2000

NW = 32
EPW = E_PAD // NW
GCHUNK = 128
NPW = 320
N_PAD = NW * NPW
CE = 4096
GB = 64


def _sc_mesh():
    return plsc.VectorSubcoreMesh(core_axis_name="c", subcore_axis_name="s")


def _wid():
    return lax.axis_index("s") * 2 + lax.axis_index("c")


def _sc_gather(A, Bx, dstp, srcp):
    H = A.shape[1]

    def body(A_hbm, B_hbm, d_hbm, s_hbm, g1_hbm, g2_hbm,
             idxd, idxs, bufA, bufB, semA, semB, semW):
        base = _wid() * EPW
        pltpu.sync_copy(d_hbm.at[pl.ds(base, EPW)], idxd)
        pltpu.sync_copy(s_hbm.at[pl.ds(base, EPW)], idxs)

        def chunk(c, carry):
            off = c * GCHUNK
            cpa = pltpu.async_copy(
                A_hbm.at[idxd.at[pl.ds(off, GCHUNK)]], bufA, semA)
            cpb = pltpu.async_copy(
                B_hbm.at[idxs.at[pl.ds(off, GCHUNK)]], bufB, semB)
            cpa.wait()
            cpb.wait()
            wa = pltpu.async_copy(bufA, g1_hbm.at[pl.ds(base + off, GCHUNK)],
                                  semW)
            wb = pltpu.async_copy(bufB, g2_hbm.at[pl.ds(base + off, GCHUNK)],
                                  semW)
            wa.wait()
            wb.wait()
            return carry

        lax.fori_loop(0, EPW // GCHUNK, chunk, 0)

    fn = pl.kernel(
        body,
        out_type=[jax.ShapeDtypeStruct((E_PAD, H), jnp.float32),
                  jax.ShapeDtypeStruct((E_PAD, H), jnp.float32)],
        mesh=_sc_mesh(),
        scratch_types=[
            pltpu.VMEM((EPW,), jnp.int32),
            pltpu.VMEM((EPW,), jnp.int32),
            pltpu.VMEM((GCHUNK, H), jnp.float32),
            pltpu.VMEM((GCHUNK, H), jnp.float32),
            pltpu.SemaphoreType.DMA,
            pltpu.SemaphoreType.DMA,
            pltpu.SemaphoreType.DMA,
        ],
    )
    return fn(A, Bx, dstp, srcp)


def _sc_scatter_max(h2, dstp, zeros_acc):
    H = h2.shape[1]

    def body(h2_hbm, d_hbm, z_hbm, out_hbm,
             ids_v, idbuf, relbuf, rowbuf, acc, semG, semZ):
        w = _wid()
        lo = w * NPW
        pltpu.async_copy(z_hbm, acc, semZ).wait()
        iota = jax.lax.iota(jnp.int32, 16)

        def chunk(c, carry):
            cbase = c * CE
            pltpu.sync_copy(d_hbm.at[pl.ds(cbase, CE)], ids_v)

            def initb(v, cc):
                idbuf[pl.ds(v * 16, 16)] = jnp.full((16,), E, jnp.int32)
                return cc

            lax.fori_loop(0, CE // 16, initb, 0)

            def scan(v, ptr):
                ids = ids_v[pl.ds(v * 16, 16)]
                m = (ids >= lo) & (ids < lo + NPW)
                mi = m.astype(jnp.int32)
                pos = ptr + plsc.cumsum(mi) - 1
                eid = iota + (cbase + v * 16)
                plsc.store_scatter(idbuf, [pos], eid, m)
                plsc.store_scatter(relbuf, [pos], ids - lo, m)
                return ptr + jnp.sum(mi)

            ptr = lax.fori_loop(0, CE // 16, scan, 0)

            def gbatch(g, carry2):
                @pl.when(g * GB < ptr)
                def _do():
                    pltpu.async_copy(
                        h2_hbm.at[idbuf.at[pl.ds(g * GB, GB)]], rowbuf,
                        semG).wait()

                    def row(r, carry3):
                        rel16 = plsc.load_gather(
                            relbuf, [jnp.full((16,), g * GB + r, jnp.int32)])
                        for hh in range(H // 16):
                            cols = iota + hh * 16
                            a = plsc.load_gather(acc, [rel16, cols])
                            xv = rowbuf[r, pl.ds(hh * 16, 16)]
                            plsc.store_scatter(acc, [rel16, cols],
                                               jnp.maximum(a, xv))
                        return carry3

                    lax.fori_loop(0, GB, row, 0)

                return carry2

            lax.fori_loop(0, CE // GB, gbatch, 0)
            return carry

        lax.fori_loop(0, E_PAD // CE, chunk, 0)
        pltpu.sync_copy(acc, out_hbm.at[pl.ds(lo, NPW)])

    fn = pl.kernel(
        body,
        out_type=jax.ShapeDtypeStruct((N_PAD, H), jnp.float32),
        mesh=_sc_mesh(),
        scratch_types=[
            pltpu.VMEM((CE,), jnp.int32),
            pltpu.VMEM((CE,), jnp.int32),
            pltpu.VMEM((CE,), jnp.int32),
            pltpu.VMEM((GB, H), jnp.float32),
            pltpu.VMEM((NPW, H), jnp.float32),
            pltpu.SemaphoreType.DMA,
            pltpu.SemaphoreType.DMA,
        ],
    )
    return fn(h2, dstp, zeros_acc)[:N]


def _edge_mm_body(H2, g1_ref, g2_ref, W2_ref, b2_ref, o_ref):
    eb = pl.program_id(0)
    h1 = jnp.maximum(g1_ref[...] + g2_ref[...], 0.0)
    h2 = lax.dot_general(h1, W2_ref[...], (((1,), (1,)), ((), ())),
                         preferred_element_type=jnp.float32)
    h2 = jnp.maximum(h2 + b2_ref[...], 0.0)
    rid = lax.broadcasted_iota(jnp.int32, (BE, H2), 0) + eb * BE
    o_ref[...] = jnp.where(rid < E, h2, 0.0)


def _edge_mm(G1, G2, W2, b2):
    H1 = G1.shape[1]
    H2 = W2.shape[0]
    return pl.pallas_call(
        functools.partial(_edge_mm_body, H2),
        grid=(E_PAD // BE,),
        in_specs=[
            pl.BlockSpec((BE, H1), lambda i: (i, 0)),
            pl.BlockSpec((BE, H1), lambda i: (i, 0)),
            pl.BlockSpec(W2.shape, lambda i: (0, 0)),
            pl.BlockSpec((1, H2), lambda i: (0, 0)),
        ],
        out_specs=pl.BlockSpec((BE, H2), lambda i: (i, 0)),
        out_shape=jax.ShapeDtypeStruct((E_PAD, H2), jnp.float32),
    )(G1, G2, W2, b2.reshape(1, -1))


def _mm_body(nparts, act, *refs):
    xs = refs[:nparts]
    Ws = refs[nparts:2 * nparts]
    b = refs[2 * nparts]
    o = refs[2 * nparts + 1]
    acc = None
    for x_ref, w_ref in zip(xs, Ws):
        p = lax.dot_general(x_ref[...], w_ref[...],
                            (((1,), (1,)), ((), ())),
                            preferred_element_type=jnp.float32)
        acc = p if acc is None else acc + p
    acc = acc + b[...]
    if act == "relu":
        acc = jnp.maximum(acc, 0.0)
    elif act == "tanh":
        acc = jnp.tanh(acc)
    o[...] = acc


def _mm(xs, Ws, b, act="relu", bn=None):
    n = xs[0].shape[0]
    bn = min(bn or BN, n)
    H = Ws[0].shape[0]
    grid = (n // bn,)
    in_specs = (
        [pl.BlockSpec((bn, x.shape[1]), lambda i: (i, 0)) for x in xs]
        + [pl.BlockSpec(W.shape, lambda i: (0, 0)) for W in Ws]
        + [pl.BlockSpec((1, H), lambda i: (0, 0))]
    )
    return pl.pallas_call(
        functools.partial(_mm_body, len(xs), act),
        grid=grid,
        in_specs=in_specs,
        out_specs=pl.BlockSpec((bn, H), lambda i: (i, 0)),
        out_shape=jax.ShapeDtypeStruct((n, H), jnp.float32),
    )(*xs, *Ws, b.reshape(1, -1))


def _edge_body(H1, H2, ids_ref, A_ref, B_ref, W2_ref, b2_ref,
               acc_ref, h1_ref, h2_ref):
    eb = pl.program_id(0)

    @pl.when(eb == 0)
    def _init():
        acc_ref[...] = jnp.zeros_like(acc_ref)

    def gather(i, carry):
        s = ids_ref[0, 0, i]
        d = ids_ref[0, 1, i]
        h1_ref[pl.ds(i, 1), :] = A_ref[pl.ds(d, 1), :] + B_ref[pl.ds(s, 1), :]
        return carry

    lax.fori_loop(0, BE, gather, 0)

    h1 = jnp.maximum(h1_ref[...], 0.0)
    h2 = lax.dot_general(h1, W2_ref[...], (((1,), (1,)), ((), ())),
                         preferred_element_type=jnp.float32)
    h2 = jnp.maximum(h2 + b2_ref[...], 0.0)
    rid = lax.broadcasted_iota(jnp.int32, (BE, H2), 0) + eb * BE
    h2_ref[...] = jnp.where(rid < E, h2, 0.0)

    def scatter(i, carry):
        d = ids_ref[0, 1, i]
        cur = acc_ref[pl.ds(d, 1), :]
        acc_ref[pl.ds(d, 1), :] = jnp.maximum(cur, h2_ref[pl.ds(i, 1), :])
        return carry

    lax.fori_loop(0, BE, scatter, 0)


def _edge_conv(ids3, A, Bx, W2, b2):
    H1 = A.shape[1]
    H2 = W2.shape[0]
    nb = ids3.shape[0]
    return pl.pallas_call(
        functools.partial(_edge_body, H1, H2),
        grid=(nb,),
        in_specs=[
            pl.BlockSpec((1, 2, BE), lambda i: (i, 0, 0),
                         memory_space=pltpu.SMEM),
            pl.BlockSpec((N, H1), lambda i: (0, 0)),
            pl.BlockSpec((N, H1), lambda i: (0, 0)),
            pl.BlockSpec(W2.shape, lambda i: (0, 0)),
            pl.BlockSpec((1, H2), lambda i: (0, 0)),
        ],
        out_specs=pl.BlockSpec((N, H2), lambda i: (0, 0)),
        out_shape=jax.ShapeDtypeStruct((N, H2), jnp.float32),
        scratch_shapes=[
            pltpu.VMEM((BE, H1), jnp.float32),
            pltpu.VMEM((BE, H2), jnp.float32),
        ],
    )(ids3, A, Bx, W2, b2.reshape(1, -1))


def _pool_body(batch_ref, x4_ref, out_ref):
    i = pl.program_id(0)

    @pl.when(i == 0)
    def _init():
        out_ref[...] = jnp.zeros_like(out_ref)

    bv = batch_ref[0]
    x4 = x4_ref[...]
    for b in range(B):
        m = (bv == b)
        mx = jnp.max(jnp.where(m, x4, 0.0), axis=0, keepdims=True)
        out_ref[pl.ds(b, 1), :] = jnp.maximum(out_ref[pl.ds(b, 1), :], mx)


def _batch_pool(batch3, x4):
    H = x4.shape[1]
    return pl.pallas_call(
        _pool_body,
        grid=(N // BN,),
        in_specs=[
            pl.BlockSpec((1, BN, 1), lambda i: (i, 0, 0)),
            pl.BlockSpec((BN, H), lambda i: (i, 0)),
        ],
        out_specs=pl.BlockSpec((B, H), lambda i: (0, 0)),
        out_shape=jax.ShapeDtypeStruct((B, H), jnp.float32),
    )(batch3, x4)


def _trans1_body(nparts, batch_ref, P_ref, *refs):
    xs = refs[:nparts]
    Ws = refs[nparts:2 * nparts]
    b = refs[2 * nparts]
    o = refs[2 * nparts + 1]
    bv = batch_ref[0]
    col = lax.broadcasted_iota(jnp.int32, (bv.shape[0], B), 1)
    oh = (bv == col).astype(jnp.float32)
    acc = lax.dot_general(oh, P_ref[...], (((1,), (0,)), ((), ())),
                          preferred_element_type=jnp.float32)
    for x_ref, w_ref in zip(xs, Ws):
        acc += lax.dot_general(x_ref[...], w_ref[...],
                               (((1,), (1,)), ((), ())),
                               preferred_element_type=jnp.float32)
    o[...] = jnp.maximum(acc + b[...], 0.0)


def _trans1(batch3, P, xs, Ws, b):
    H = P.shape[1]
    in_specs = (
        [pl.BlockSpec((1, BN, 1), lambda i: (i, 0, 0)),
         pl.BlockSpec(P.shape, lambda i: (0, 0))]
        + [pl.BlockSpec((BN, x.shape[1]), lambda i: (i, 0)) for x in xs]
        + [pl.BlockSpec(W.shape, lambda i: (0, 0)) for W in Ws]
        + [pl.BlockSpec((1, H), lambda i: (0, 0))]
    )
    return pl.pallas_call(
        functools.partial(_trans1_body, len(xs)),
        grid=(N // BN,),
        in_specs=in_specs,
        out_specs=pl.BlockSpec((BN, H), lambda i: (i, 0)),
        out_shape=jax.ShapeDtypeStruct((N, H), jnp.float32),
    )(batch3, P, *xs, *Ws, b.reshape(1, -1))


def _gcu(x, ids3_tpl, ids3_geo, tpl_p, geo_p, mlp_p):
    outs = []
    for ids3, (l1, l2) in ((ids3_tpl, tpl_p), (ids3_geo, geo_p)):
        W1, b1 = l1
        W2, b2 = l2
        half = W1.shape[1] // 2
        W1a = W1[:, :half]
        W1b = W1[:, half:]
        A = _mm([x], [W1a - W1b], b1, act=None)
        Bx = _mm([x], [W1b], jnp.zeros_like(b1), act=None)
        outs.append(_edge_conv(ids3, A, Bx, W2, b2))
    Wm, bm = mlp_p[0]
    half = Wm.shape[1] // 2
    return _mm(outs, [Wm[:, :half], Wm[:, half:]], bm)


def kernel(pos, x, tpl_edge_index, geo_edge_index, batch,
           g1_tpl, g1_geo, g1_mlp, g2_tpl, g2_geo, g2_mlp,
           g3_tpl, g3_geo, g3_mlp, glb, trans_mlp, trans_w, trans_b):
    xin = jnp.concatenate(
        [pos, x, jnp.zeros((N, 10), jnp.float32)], axis=1)

    def prep_ids(ei):
        p = jnp.concatenate(
            [ei.astype(jnp.int32),
             jnp.zeros((2, E_PAD - E), jnp.int32)], axis=1)
        return p.reshape(2, E_PAD // BE, BE).transpose(1, 0, 2)

    ids3_tpl = prep_ids(tpl_edge_index)
    ids3_geo = prep_ids(geo_edge_index)
    batch3 = batch.astype(jnp.int32).reshape(N // BN, BN, 1)

    def pad_cols(W, k):
        return jnp.concatenate(
            [W, jnp.zeros((W.shape[0], k - W.shape[1]), W.dtype)], axis=1)

    def g1_prep(p):
        (W1, b1), l2 = p
        half = W1.shape[1] // 2
        W1p = jnp.concatenate(
            [pad_cols(W1[:, :half], 16), pad_cols(W1[:, half:], 16)], axis=1)
        return [(W1p, b1), l2]

    x1 = _gcu(xin, ids3_tpl, ids3_geo, g1_prep(g1_tpl), g1_prep(g1_geo),
              g1_mlp)
    x2 = _gcu(x1, ids3_tpl, ids3_geo, g2_tpl, g2_geo, g2_mlp)
    x3 = _gcu(x2, ids3_tpl, ids3_geo, g3_tpl, g3_geo, g3_mlp)

    Wg, bg = glb[0]
    x4 = _mm([x1, x2, x3], [Wg[:, :64], Wg[:, 64:320], Wg[:, 320:]], bg)

    xg = _batch_pool(batch3, x4)

    (Wt1, bt1), (Wt2, bt2) = trans_mlp
    P = _mm([xg], [Wt1[:, :1024]], jnp.zeros_like(bt1), act=None, bn=B)
    y1 = _trans1(
        batch3, P,
        [xin, x1, x2, x3],
        [pad_cols(Wt1[:, 1024:1030], 16), Wt1[:, 1030:1094],
         Wt1[:, 1094:1350], Wt1[:, 1350:1862]],
        bt1)
    y2 = _mm([y1], [Wt2], bt2)
    Wf = jnp.concatenate(
        [trans_w, jnp.zeros((128 - trans_w.shape[0], 256), jnp.float32)],
        axis=0)
    bf = jnp.concatenate(
        [trans_b, jnp.zeros((128 - trans_b.shape[0],), jnp.float32)])
    out = _mm([y2], [Wf], bf, act="tanh")
    return out[:, :3]
